# Initial kernel scaffold; baseline (speedup 1.0000x reference)
import functools

import jax
import jax.numpy as jnp
from jax import lax
from jax.experimental import pallas as pl
from jax.experimental.pallas import tpu as pltpu
from jax.experimental.pallas import tpu_sc as plsc

N_NODES_C = 10000
N_EDGES_C = 320000
D_C = 128

_CHUNK = 128
_NCHUNK = N_EDGES_C // _CHUNK
_NC, _NS = 2, 16
_NW = _NC * _NS
_FULL = _NCHUNK // _NW
_REM = _NCHUNK - _FULL * _NW
_ROWS_PER_TILE = N_NODES_C // _NS


def _sc_scatter_body(feat_hbm, src_hbm, dst_hbm, out_hbm,
                     idx_src, idx_dst, rows, hacc, sem):
    cid = lax.axis_index("c")
    sid = lax.axis_index("s")
    wid = sid * _NC + cid

    def _zero_row(r, _):
        for g in range(D_C // 16):
            rows[r, pl.ds(g * 16, 16)] = jnp.zeros((16,), jnp.float32)
        return 0
    lax.fori_loop(0, _CHUNK, _zero_row, 0)
    base_row = sid * _ROWS_PER_TILE
    for k in range(5):
        pltpu.sync_copy(rows.at[pl.ds(0, 125)],
                        hacc.at[pl.ds(base_row + k * 125, 125)])
    plsc.subcore_barrier()

    n_iter = jnp.where(wid < _REM, _FULL + 1, _FULL)

    def _chunk(j, _):
        c = j * _NW + wid
        pltpu.sync_copy(src_hbm.at[c], idx_src)
        pltpu.sync_copy(dst_hbm.at[c], idx_dst)
        pltpu.async_copy(feat_hbm.at[idx_src], rows, sem).wait()
        pltpu.sync_copy(rows, hacc.at[idx_dst], add=True)
        return 0
    lax.fori_loop(0, n_iter, _chunk, 0)
    plsc.subcore_barrier()

    pltpu.sync_copy(hacc.at[pl.ds(base_row, _ROWS_PER_TILE)],
                    out_hbm.at[cid, pl.ds(base_row, _ROWS_PER_TILE)])


def _sc_scatter(feature, src2d, dst2d):
    mesh = plsc.VectorSubcoreMesh(core_axis_name="c", subcore_axis_name="s")
    return pl.kernel(
        _sc_scatter_body,
        out_type=jax.ShapeDtypeStruct((_NC, N_NODES_C, D_C), jnp.float32),
        mesh=mesh,
        scratch_types=[
            pltpu.VMEM((_CHUNK,), jnp.int32),
            pltpu.VMEM((_CHUNK,), jnp.int32),
            pltpu.VMEM((_CHUNK, D_C), jnp.float32),
            pltpu.VMEM_SHARED((N_NODES_C, D_C), jnp.float32),
            pltpu.SemaphoreType.DMA,
        ],
    )(feature, src2d, dst2d)


def _tc_linear_body(p0_ref, p1_ref, w_ref, b_ref, out_ref):
    acc = p0_ref[...] + p1_ref[...]
    out_ref[...] = lax.dot_general(
        acc, w_ref[...], (((1,), (1,)), ((), ())),
        preferred_element_type=jnp.float32) + b_ref[...]


def _tc_linear(p0, p1, W, b2d):
    br = 2000
    grid = (N_NODES_C // br,)
    return pl.pallas_call(
        _tc_linear_body,
        grid=grid,
        in_specs=[
            pl.BlockSpec((br, D_C), lambda i: (i, 0)),
            pl.BlockSpec((br, D_C), lambda i: (i, 0)),
            pl.BlockSpec((D_C, D_C), lambda i: (0, 0)),
            pl.BlockSpec((1, D_C), lambda i: (0, 0)),
        ],
        out_specs=pl.BlockSpec((br, D_C), lambda i: (i, 0)),
        out_shape=jax.ShapeDtypeStruct((N_NODES_C, D_C), jnp.float32),
    )(p0, p1, W, b2d)


def kernel(feature, edge_index, W, b):
    ei = edge_index.astype(jnp.int32)
    src2d = ei[0].reshape(_NCHUNK, _CHUNK)
    dst2d = ei[1].reshape(_NCHUNK, _CHUNK)
    partial = _sc_scatter(feature, src2d, dst2d)
    return _tc_linear(partial[0], partial[1], W, b.reshape(1, D_C))


# speedup vs baseline: 6.6923x; 6.6923x over previous
import functools

import jax
import jax.numpy as jnp
from jax import lax
from jax.experimental import pallas as pl
from jax.experimental.pallas import tpu as pltpu
from jax.experimental.pallas import tpu_sc as plsc

N_NODES_C = 10000
N_EDGES_C = 320000
D_C = 128

_CHUNK = 128
_NCHUNK = N_EDGES_C // _CHUNK
_NC, _NS = 2, 16
_NW = _NC * _NS
_FULL = _NCHUNK // _NW
_REM = _NCHUNK - _FULL * _NW
_ROWS_PER_TILE = N_NODES_C // _NS


def _sc_scatter_body(feat_hbm, src_hbm, dst_hbm, out_hbm,
                     idx_src, idx_dst, rows, hacc, sem):
    cid = lax.axis_index("c")
    sid = lax.axis_index("s")
    wid = sid * _NC + cid

    def _zero_row(r, _):
        for g in range(D_C // 16):
            rows[r, pl.ds(g * 16, 16)] = jnp.zeros((16,), jnp.float32)
        return 0
    lax.fori_loop(0, _CHUNK, _zero_row, 0)
    base_row = sid * _ROWS_PER_TILE
    for k in range(5):
        pltpu.sync_copy(rows.at[pl.ds(0, 125)],
                        hacc.at[pl.ds(base_row + k * 125, 125)])
    plsc.subcore_barrier()

    n_iter = jnp.where(wid < _REM, _FULL + 1, _FULL)

    def _chunk(j, _):
        c = j * _NW + wid
        pltpu.sync_copy(src_hbm.at[c], idx_src)
        pltpu.sync_copy(dst_hbm.at[c], idx_dst)
        pltpu.async_copy(feat_hbm.at[idx_src], rows, sem).wait()
        pltpu.sync_copy(rows, hacc.at[idx_dst], add=True)
        return 0
    lax.fori_loop(0, n_iter, _chunk, 0)
    plsc.subcore_barrier()

    pltpu.sync_copy(hacc.at[pl.ds(base_row, _ROWS_PER_TILE)],
                    out_hbm.at[cid, pl.ds(base_row, _ROWS_PER_TILE)])


def _sc_scatter(feature, src2d, dst2d):
    mesh = plsc.VectorSubcoreMesh(core_axis_name="c", subcore_axis_name="s")
    return pl.kernel(
        _sc_scatter_body,
        out_type=jax.ShapeDtypeStruct((_NC, N_NODES_C, D_C), jnp.float32),
        mesh=mesh,
        scratch_types=[
            pltpu.VMEM((_CHUNK,), jnp.int32),
            pltpu.VMEM((_CHUNK,), jnp.int32),
            pltpu.VMEM((_CHUNK, D_C), jnp.float32),
            pltpu.VMEM_SHARED((N_NODES_C, D_C), jnp.float32),
            pltpu.SemaphoreType.DMA,
        ],
        compiler_params=pltpu.CompilerParams(use_tc_tiling_on_sc=False),
    )(feature, src2d, dst2d)


def _tc_linear_body(p0_ref, p1_ref, w_ref, b_ref, out_ref):
    acc = p0_ref[...] + p1_ref[...]
    out_ref[...] = lax.dot_general(
        acc, w_ref[...], (((1,), (1,)), ((), ())),
        preferred_element_type=jnp.float32) + b_ref[...]


def _tc_linear(p0, p1, W, b2d):
    br = 2000
    grid = (N_NODES_C // br,)
    return pl.pallas_call(
        _tc_linear_body,
        grid=grid,
        in_specs=[
            pl.BlockSpec((br, D_C), lambda i: (i, 0)),
            pl.BlockSpec((br, D_C), lambda i: (i, 0)),
            pl.BlockSpec((D_C, D_C), lambda i: (0, 0)),
            pl.BlockSpec((1, D_C), lambda i: (0, 0)),
        ],
        out_specs=pl.BlockSpec((br, D_C), lambda i: (i, 0)),
        out_shape=jax.ShapeDtypeStruct((N_NODES_C, D_C), jnp.float32),
    )(p0, p1, W, b2d)


def kernel(feature, edge_index, W, b):
    ei = edge_index.astype(jnp.int32)
    src2d = ei[0].reshape(_NCHUNK, _CHUNK)
    dst2d = ei[1].reshape(_NCHUNK, _CHUNK)
    partial = _sc_scatter(feature, src2d, dst2d)
    return _tc_linear(partial[0], partial[1], W, b.reshape(1, D_C))
